# baseline (speedup 1.0000x reference)
import functools

import numpy as np
import jax
import jax.numpy as jnp
from jax import lax
from jax.experimental import pallas as pl
from jax.experimental.pallas import tpu as pltpu
from jax.experimental.pallas import tpu_sc as plsc

_NPERSEG = 32
_STEP = 16
_NSEG = 31
_NF = 17
_L = 512
_NREF = 16
_C = 8
_NB = 512

_HIGH = jax.lax.Precision.HIGHEST
_DNT = (((1,), (1,)), ((), ()))


def _stft_ri_t(x):
    step = _NPERSEG - _NPERSEG // 2
    L = x.shape[-1]
    nseg = (L - _NPERSEG) // step + 1
    idx = np.arange(nseg)[:, None] * step + np.arange(_NPERSEG)[None, :]
    frames = x[..., idx]
    frames = frames - jnp.mean(frames, axis=-1, keepdims=True)
    n = np.arange(_NPERSEG)
    win = (0.5 - 0.5 * np.cos(2.0 * np.pi * n / _NPERSEG)).astype(np.float32)
    Z = jnp.fft.rfft(frames * win, axis=-1)
    Zr = jnp.real(Z).astype(jnp.float32)
    Zi = jnp.imag(Z).astype(jnp.float32)
    Zr = jnp.moveaxis(Zr, -1, -3)
    Zi = jnp.moveaxis(Zi, -1, -3)
    return Zr, Zi


def _dot_d(a, b):
    return jax.lax.dot_general(a, b, _DNT, preferred_element_type=jnp.float32)


def _dot_h(a, b):
    return jax.lax.dot_general(a, b, _DNT, precision=_HIGH,
                               preferred_element_type=jnp.float32)


def _scores_body(xr_ref, xi_ref, yr_ref, yi_ref, out_ref):
    nb = yr_ref.shape[2]
    ones = jnp.ones((1, _NSEG), jnp.float32)
    score = jnp.zeros((nb, _C), jnp.float32)
    for f in range(_NF):
        xr = xr_ref[0, f]
        xi = xi_ref[0, f]
        yr = yr_ref[0, f]
        yi = yi_ref[0, f]
        Pre = _dot_d(yr, xr) + _dot_d(yi, xi)
        Pim = _dot_d(yr, xi) - _dot_d(yi, xr)
        Pxx = _dot_h(ones, xr * xr + xi * xi)
        Pyy = jnp.sum(yr * yr + yi * yi, axis=1, keepdims=True)
        numer = (Pre * Pre + Pim * Pim) * np.float32(1.0 / (_NSEG * _NSEG))
        denom = (Pxx * np.float32(1.0 / _NSEG)) * (Pyy * np.float32(1.0 / _NSEG)) \
            + np.float32(1e-12)
        score = score + numer / denom
    out_ref[0] = score * np.float32(1.0 / _NF)


def _topk_body(s_ref, idx_ref):
    s = s_ref[0]
    n = s.shape[0]
    iota0 = lax.broadcasted_iota(jnp.int32, (n, _C), 0)
    rows16 = lax.broadcasted_iota(jnp.int32, (_NREF, _C), 0)
    acc = jnp.zeros((_NREF, _C), jnp.int32)
    for t in range(_NREF):
        m = jnp.max(s, axis=0, keepdims=True)
        cand = jnp.where(s == m, iota0, jnp.int32(2 * n))
        first = jnp.min(cand, axis=0, keepdims=True)
        acc = jnp.where(rows16 == t, jnp.broadcast_to(first, (_NREF, _C)), acc)
        s = jnp.where(iota0 == first, -jnp.inf, s)
    idx_ref[0] = acc


def _sc_gather(dbflat, idx_flat):
    info = plsc.get_sparse_core_info()
    nw = info.num_cores * info.num_subcores
    nrows = idx_flat.shape[0]
    b_per_w = nrows // nw
    mesh = plsc.VectorSubcoreMesh(core_axis_name="c", subcore_axis_name="s")

    @functools.partial(
        pl.kernel, mesh=mesh,
        out_type=jax.ShapeDtypeStruct((nrows, _L), jnp.float32),
        scratch_types=[
            pltpu.VMEM((b_per_w,), jnp.int32),
            pltpu.VMEM((b_per_w, _L), jnp.float32),
            pltpu.SemaphoreType.DMA,
        ],
    )
    def gather_k(table_hbm, idx_hbm, out_hbm, idx_v, rows_v, sem):
        wid = lax.axis_index("s") * info.num_cores + lax.axis_index("c")
        base = wid * b_per_w
        pltpu.sync_copy(idx_hbm.at[pl.ds(base, b_per_w)], idx_v)
        pltpu.async_copy(table_hbm.at[idx_v], rows_v, sem).wait()
        pltpu.sync_copy(rows_v, out_hbm.at[pl.ds(base, b_per_w)])

    return gather_k(dbflat, idx_flat)


def _compute_scores(target_series, TS_database):
    B, C, L = target_series.shape
    N = TS_database.shape[1]
    Xr, Xi = _stft_ri_t(target_series)
    Yr, Yi = _stft_ri_t(TS_database)
    return pl.pallas_call(
        _scores_body,
        grid=(B, N // _NB),
        in_specs=[
            pl.BlockSpec((1, _NF, C, _NSEG), lambda b, j: (b, 0, 0, 0)),
            pl.BlockSpec((1, _NF, C, _NSEG), lambda b, j: (b, 0, 0, 0)),
            pl.BlockSpec((1, _NF, _NB, _NSEG), lambda b, j: (b, 0, j, 0)),
            pl.BlockSpec((1, _NF, _NB, _NSEG), lambda b, j: (b, 0, j, 0)),
        ],
        out_specs=pl.BlockSpec((1, _NB, C), lambda b, j: (b, j, 0)),
        out_shape=jax.ShapeDtypeStruct((B, N, C), jnp.float32),
    )(Xr, Xi, Yr, Yi)


def _compute_topk(scores):
    B, N, C = scores.shape
    return pl.pallas_call(
        _topk_body,
        grid=(B,),
        in_specs=[pl.BlockSpec((1, N, C), lambda b: (b, 0, 0))],
        out_specs=pl.BlockSpec((1, _NREF, C), lambda b: (b, 0, 0)),
        out_shape=jax.ShapeDtypeStruct((B, _NREF, C), jnp.int32),
    )(scores)


def kernel(target_series, TS_database):
    B, C, L = target_series.shape
    N = TS_database.shape[1]
    scores = _compute_scores(target_series, TS_database)
    idx = _compute_topk(scores)
    idx_flat = (idx + (jnp.arange(B, dtype=jnp.int32) * N)[:, None, None])
    idx_flat = idx_flat.transpose(0, 2, 1).reshape(B * C * _NREF)
    rows = _sc_gather(TS_database.reshape(B * N, L), idx_flat)
    return rows.reshape(B, C * _NREF, L)
